# baseline (speedup 1.0000x reference)
import functools

import jax
import jax.numpy as jnp
from jax import lax
from jax.experimental import pallas as pl
from jax.experimental.pallas import tpu as pltpu
from jax.experimental.pallas import tpu_sc as plsc

VOCAB = 100000
HIDDEN = 2048
BATCH = 4
SEQ = 2048
B = BATCH * SEQ

TC_N = 1024
SC_N = B - TC_N

NUM_CORES = 2
NUM_SUBCORES = 16
NW = NUM_CORES * NUM_SUBCORES
BPW = SC_N // NW
CHUNK = 16
NCHUNK = BPW // CHUNK
NBUF = 3

TC_G = 64
TC_NG = TC_N // TC_G


def _emb_sc_kernel(idx_hbm, table_hbm, out_hbm, idx_v, rows_v, gsem, ssem):
    wid = lax.axis_index("s") * NUM_CORES + lax.axis_index("c")
    base = wid * BPW
    pltpu.sync_copy(idx_hbm.at[pl.ds(base, BPW)], idx_v)

    def issue_g(ch, buf):
        return pltpu.async_copy(
            table_hbm.at[idx_v.at[pl.ds(ch * CHUNK, CHUNK)]],
            rows_v.at[buf],
            gsem,
        )

    def issue_s(ch, buf):
        return pltpu.async_copy(
            rows_v.at[buf],
            out_hbm.at[pl.ds(base + ch * CHUNK, CHUNK)],
            ssem,
        )

    g = {0: issue_g(0, 0), 1: issue_g(1, 1)}
    s = {}
    for ch in range(NCHUNK):
        g[ch].wait()
        s[ch] = issue_s(ch, ch % NBUF)
        nxt = ch + 2
        if nxt < NCHUNK:
            if nxt - NBUF >= 0:
                s[nxt - NBUF].wait()
            g[nxt] = issue_g(nxt, nxt % NBUF)
    for j in range(max(0, NCHUNK - NBUF), NCHUNK):
        s[j].wait()


def _emb_tc_kernel(idx_sm, table_hbm, out_hbm, buf, gsem, osem):
    wb = {}
    for grp in range(TC_NG):
        b = grp % 2
        if grp >= 2:
            wb[grp - 2].wait()
        hs = []
        for i in range(TC_G):
            row = idx_sm[grp * TC_G + i]
            c = pltpu.make_async_copy(
                table_hbm.at[pl.ds(row, 1)],
                buf.at[b, pl.ds(i, 1)],
                gsem,
            )
            c.start()
            hs.append(c)
        for c in hs:
            c.wait()
        w = pltpu.make_async_copy(
            buf.at[b],
            out_hbm.at[pl.ds(grp * TC_G, TC_G)],
            osem,
        )
        w.start()
        wb[grp] = w
    wb[TC_NG - 2].wait()
    wb[TC_NG - 1].wait()


@jax.jit
def _emb(idx_flat, table):
    mesh = plsc.VectorSubcoreMesh(core_axis_name="c", subcore_axis_name="s")
    sc = functools.partial(
        pl.kernel,
        mesh=mesh,
        out_type=jax.ShapeDtypeStruct((B, HIDDEN), jnp.float32),
        scratch_types=[
            pltpu.VMEM((BPW,), jnp.int32),
            pltpu.VMEM((NBUF, CHUNK, HIDDEN), jnp.float32),
            pltpu.SemaphoreType.DMA,
            pltpu.SemaphoreType.DMA,
        ],
    )(_emb_sc_kernel)
    out_sc = sc(idx_flat, table)

    tc_rows = pl.pallas_call(
        _emb_tc_kernel,
        grid=(),
        in_specs=[
            pl.BlockSpec(memory_space=pltpu.SMEM),
            pl.BlockSpec(memory_space=pl.ANY),
        ],
        out_specs=pl.BlockSpec(memory_space=pl.ANY),
        out_shape=jax.ShapeDtypeStruct((TC_N, HIDDEN), jnp.float32),
        scratch_shapes=[
            pltpu.VMEM((2, TC_G, HIDDEN), jnp.float32),
            pltpu.SemaphoreType.DMA,
            pltpu.SemaphoreType.DMA,
        ],
    )(idx_flat[SC_N:], table)

    return lax.dynamic_update_slice(out_sc, tc_rows, (SC_N, 0))


def kernel(token_ids, embed_weight):
    batch, seq = token_ids.shape
    idx_flat = token_ids.reshape(-1).astype(jnp.int32)
    out = _emb(idx_flat, embed_weight)
    return out.reshape(batch, seq, HIDDEN)


# speedup vs baseline: 1.0970x; 1.0970x over previous
import functools

import jax
import jax.numpy as jnp
from jax import lax
from jax.experimental import pallas as pl
from jax.experimental.pallas import tpu as pltpu
from jax.experimental.pallas import tpu_sc as plsc

VOCAB = 100000
HIDDEN = 2048
BATCH = 4
SEQ = 2048
B = BATCH * SEQ

NUM_CORES = 2
NUM_SUBCORES = 16
NW = NUM_CORES * NUM_SUBCORES
BPW = B // NW
CHUNK = 16
NCHUNK = BPW // CHUNK
NBUF = 3
WPB = SEQ // BPW


def _emb_kernel(idx_hbm, table_hbm, out_hbm, idx_v, rows_v, gsem, ssem):
    wid = lax.axis_index("s") * NUM_CORES + lax.axis_index("c")
    b0 = wid // WPB
    col = (wid % WPB) * BPW
    pltpu.sync_copy(idx_hbm.at[b0, pl.ds(col, BPW)], idx_v)

    def issue_g(ch):
        return pltpu.async_copy(
            table_hbm.at[idx_v.at[pl.ds(ch * CHUNK, CHUNK)]],
            rows_v.at[ch % NBUF],
            gsem,
        )

    def issue_s(ch):
        return pltpu.async_copy(
            rows_v.at[ch % NBUF],
            out_hbm.at[b0, pl.ds(col + ch * CHUNK, CHUNK)],
            ssem,
        )

    g = {0: issue_g(0), 1: issue_g(1)}
    s = {}
    for ch in range(NCHUNK):
        g[ch].wait()
        s[ch] = issue_s(ch)
        nxt = ch + 2
        if nxt < NCHUNK:
            if nxt - NBUF >= 0:
                s[nxt - NBUF].wait()
            g[nxt] = issue_g(nxt)
    for j in range(max(0, NCHUNK - NBUF), NCHUNK):
        s[j].wait()


@jax.jit
def _emb(token_ids, table):
    mesh = plsc.VectorSubcoreMesh(core_axis_name="c", subcore_axis_name="s")
    f = functools.partial(
        pl.kernel,
        mesh=mesh,
        out_type=jax.ShapeDtypeStruct((BATCH, SEQ, HIDDEN), jnp.float32),
        scratch_types=[
            pltpu.VMEM((BPW,), jnp.int32),
            pltpu.VMEM((NBUF, CHUNK, HIDDEN), jnp.float32),
            pltpu.SemaphoreType.DMA,
            pltpu.SemaphoreType.DMA,
        ],
    )(_emb_kernel)
    return f(token_ids, table)


def kernel(token_ids, embed_weight):
    return _emb(token_ids.astype(jnp.int32), embed_weight)
